# baseline (speedup 1.0000x reference)
import functools

import jax
import jax.numpy as jnp
from jax.experimental import pallas as pl
from jax.experimental.pallas import tpu as pltpu

EMB = 256
IN = 2 * EMB


def _body(nj, adj_ref, a_ref, h0_ref, wagg_ref, bagg_ref, wfcra_ref,
          bfcra_ref, out_ref, acc_ref, deg_ref, h_ref):
    k = pl.program_id(1)
    j = pl.program_id(2)
    depth = pl.num_programs(1)

    @pl.when(jnp.logical_and(k == 0, j == 0))
    def _():
        h_ref[...] = h0_ref[...]

    @pl.when(j == 0)
    def _():
        acc_ref[...] = jnp.zeros_like(acc_ref)
        deg_ref[...] = jnp.zeros_like(deg_ref)

    adj_blk = adj_ref[0]
    acc_ref[...] += jnp.dot(adj_blk, a_ref[0], preferred_element_type=jnp.float32)
    deg_ref[...] += jnp.sum(adj_blk, axis=1, keepdims=True)

    @pl.when(j == nj - 1)
    def _():
        deg = jnp.maximum(deg_ref[...], 1e-6)
        agg = acc_ref[...] / deg
        emb = jnp.dot(agg, wagg_ref[0], preferred_element_type=jnp.float32)
        emb = jnp.maximum(emb + bagg_ref[0], 0.0)
        wf = wfcra_ref[0]
        h_prev = h_ref[...]
        h_new = (jnp.dot(emb, wf[:EMB], preferred_element_type=jnp.float32)
                 + jnp.dot(h_prev, wf[EMB:], preferred_element_type=jnp.float32)
                 + bfcra_ref[0])
        h_new = jnp.maximum(h_new, 0.0)
        h_ref[...] = h_new

        @pl.when(k == depth - 1)
        def _():
            out_ref[...] = h_new


def kernel(h0, a, adjacent_mat, W_agg, b_agg, W_fcra, b_fcra):
    n = h0.shape[0]
    depth = a.shape[0]
    tm, tk = 256, 1024
    ni, nj = n // tm, n // tk
    b_agg3 = b_agg.reshape(depth, 1, EMB)
    b_fcra3 = b_fcra.reshape(depth, 1, EMB)

    grid = (ni, depth, nj)
    out = pl.pallas_call(
        functools.partial(_body, nj),
        grid=grid,
        in_specs=[
            pl.BlockSpec((1, tm, tk), lambda i, k, j: (k, i, j)),
            pl.BlockSpec((1, tk, IN), lambda i, k, j: (k, j, 0)),
            pl.BlockSpec((tm, EMB), lambda i, k, j: (i, 0)),
            pl.BlockSpec((1, IN, EMB), lambda i, k, j: (k, 0, 0)),
            pl.BlockSpec((1, 1, EMB), lambda i, k, j: (k, 0, 0)),
            pl.BlockSpec((1, IN, EMB), lambda i, k, j: (k, 0, 0)),
            pl.BlockSpec((1, 1, EMB), lambda i, k, j: (k, 0, 0)),
        ],
        out_specs=pl.BlockSpec((tm, EMB), lambda i, k, j: (i, 0)),
        out_shape=jax.ShapeDtypeStruct((n, EMB), jnp.float32),
        scratch_shapes=[
            pltpu.VMEM((tm, IN), jnp.float32),
            pltpu.VMEM((tm, 1), jnp.float32),
            pltpu.VMEM((tm, EMB), jnp.float32),
        ],
        compiler_params=pltpu.CompilerParams(
            dimension_semantics=("arbitrary", "arbitrary", "arbitrary"),
        ),
    )(adjacent_mat, a, h0, W_agg, b_agg3, W_fcra, b_fcra3)
    return out


# speedup vs baseline: 1.9964x; 1.9964x over previous
import functools

import jax
import jax.numpy as jnp
from jax.experimental import pallas as pl
from jax.experimental.pallas import tpu as pltpu

EMB = 256
IN = 2 * EMB


def _body(nj, adj_ref, a_ref, h0_ref, wagg_ref, bagg_ref, wfcra_ref,
          bfcra_ref, out_ref, acc_ref, deg_ref, h_ref):
    k = pl.program_id(0)
    j = pl.program_id(1)
    depth = pl.num_programs(0)

    @pl.when(jnp.logical_and(k == 0, j == 0))
    def _():
        h_ref[...] = h0_ref[...]

    @pl.when(j == 0)
    def _():
        acc_ref[...] = jnp.zeros_like(acc_ref)
        deg_ref[...] = jnp.zeros_like(deg_ref)

    adj_blk = adj_ref[0]
    acc_ref[...] += jnp.dot(adj_blk, a_ref[0], preferred_element_type=jnp.float32)
    deg_ref[...] += jnp.sum(adj_blk, axis=1, keepdims=True)

    @pl.when(j == nj - 1)
    def _():
        deg_inv = 1.0 / jnp.maximum(deg_ref[...], 1e-6)
        agg = acc_ref[...] * deg_inv
        emb = jnp.dot(agg, wagg_ref[0], preferred_element_type=jnp.float32)
        emb = jnp.maximum(emb + bagg_ref[0], 0.0)
        wf = wfcra_ref[0]
        h_prev = h_ref[...]
        h_new = (jnp.dot(emb, wf[:EMB], preferred_element_type=jnp.float32)
                 + jnp.dot(h_prev, wf[EMB:], preferred_element_type=jnp.float32)
                 + bfcra_ref[0])
        h_new = jnp.maximum(h_new, 0.0)
        h_ref[...] = h_new

        @pl.when(k == depth - 1)
        def _():
            out_ref[...] = h_new


def kernel(h0, a, adjacent_mat, W_agg, b_agg, W_fcra, b_fcra):
    n = h0.shape[0]
    depth = a.shape[0]
    tk = 512
    nj = n // tk
    b_agg3 = b_agg.reshape(depth, 1, EMB)
    b_fcra3 = b_fcra.reshape(depth, 1, EMB)

    grid = (depth, nj)
    out = pl.pallas_call(
        functools.partial(_body, nj),
        grid=grid,
        in_specs=[
            pl.BlockSpec((1, n, tk), lambda k, j: (k, 0, j)),
            pl.BlockSpec((1, tk, IN), lambda k, j: (k, j, 0)),
            pl.BlockSpec((n, EMB), lambda k, j: (0, 0)),
            pl.BlockSpec((1, IN, EMB), lambda k, j: (k, 0, 0)),
            pl.BlockSpec((1, 1, EMB), lambda k, j: (k, 0, 0)),
            pl.BlockSpec((1, IN, EMB), lambda k, j: (k, 0, 0)),
            pl.BlockSpec((1, 1, EMB), lambda k, j: (k, 0, 0)),
        ],
        out_specs=pl.BlockSpec((n, EMB), lambda k, j: (0, 0)),
        out_shape=jax.ShapeDtypeStruct((n, EMB), jnp.float32),
        scratch_shapes=[
            pltpu.VMEM((n, IN), jnp.float32),
            pltpu.VMEM((n, 1), jnp.float32),
            pltpu.VMEM((n, EMB), jnp.float32),
        ],
        compiler_params=pltpu.CompilerParams(
            dimension_semantics=("arbitrary", "arbitrary"),
        ),
    )(adjacent_mat, a, h0, W_agg, b_agg3, W_fcra, b_fcra3)
    return out


# speedup vs baseline: 2.0182x; 1.0109x over previous
import functools

import jax
import jax.numpy as jnp
from jax.experimental import pallas as pl
from jax.experimental.pallas import tpu as pltpu

EMB = 256
IN = 2 * EMB


def _body(nj, adj_ref, a_ref, h0_ref, wagg_ref, bagg_ref, wfcra_ref,
          bfcra_ref, out_ref, acc_ref, deg_ref, h_ref):
    k = pl.program_id(0)
    j = pl.program_id(1)
    depth = pl.num_programs(0)

    @pl.when(jnp.logical_and(k == 0, j == 0))
    def _():
        h_ref[...] = h0_ref[...]

    @pl.when(j == 0)
    def _():
        acc_ref[...] = jnp.zeros_like(acc_ref)
        deg_ref[...] = jnp.zeros_like(deg_ref)

    adj_blk = adj_ref[0]
    acc_ref[...] += jnp.dot(adj_blk.astype(jnp.bfloat16),
                            a_ref[0].astype(jnp.bfloat16),
                            preferred_element_type=jnp.float32)
    deg_ref[...] += jnp.sum(adj_blk, axis=1, keepdims=True)

    @pl.when(j == nj - 1)
    def _():
        deg_inv = 1.0 / jnp.maximum(deg_ref[...], 1e-6)
        agg = (acc_ref[...] * deg_inv).astype(jnp.bfloat16)
        wagg = wagg_ref[0].astype(jnp.bfloat16)
        emb = jnp.dot(agg, wagg, preferred_element_type=jnp.float32)
        emb = jnp.maximum(emb + bagg_ref[0], 0.0).astype(jnp.bfloat16)
        wf = wfcra_ref[0].astype(jnp.bfloat16)
        h_prev = h_ref[...].astype(jnp.bfloat16)
        h_new = (jnp.dot(emb, wf[:EMB], preferred_element_type=jnp.float32)
                 + jnp.dot(h_prev, wf[EMB:], preferred_element_type=jnp.float32)
                 + bfcra_ref[0])
        h_new = jnp.maximum(h_new, 0.0)
        h_ref[...] = h_new

        @pl.when(k == depth - 1)
        def _():
            out_ref[...] = h_new


def kernel(h0, a, adjacent_mat, W_agg, b_agg, W_fcra, b_fcra):
    n = h0.shape[0]
    depth = a.shape[0]
    tk = 512
    nj = n // tk
    b_agg3 = b_agg.reshape(depth, 1, EMB)
    b_fcra3 = b_fcra.reshape(depth, 1, EMB)

    grid = (depth, nj)
    out = pl.pallas_call(
        functools.partial(_body, nj),
        grid=grid,
        in_specs=[
            pl.BlockSpec((1, n, tk), lambda k, j: (k, 0, j)),
            pl.BlockSpec((1, tk, IN), lambda k, j: (k, j, 0)),
            pl.BlockSpec((n, EMB), lambda k, j: (0, 0)),
            pl.BlockSpec((1, IN, EMB), lambda k, j: (k, 0, 0)),
            pl.BlockSpec((1, 1, EMB), lambda k, j: (k, 0, 0)),
            pl.BlockSpec((1, IN, EMB), lambda k, j: (k, 0, 0)),
            pl.BlockSpec((1, 1, EMB), lambda k, j: (k, 0, 0)),
        ],
        out_specs=pl.BlockSpec((n, EMB), lambda k, j: (0, 0)),
        out_shape=jax.ShapeDtypeStruct((n, EMB), jnp.float32),
        scratch_shapes=[
            pltpu.VMEM((n, IN), jnp.float32),
            pltpu.VMEM((n, 1), jnp.float32),
            pltpu.VMEM((n, EMB), jnp.float32),
        ],
        compiler_params=pltpu.CompilerParams(
            dimension_semantics=("arbitrary", "arbitrary"),
        ),
    )(adjacent_mat, a, h0, W_agg, b_agg3, W_fcra, b_fcra3)
    return out
